# Initial kernel scaffold; baseline (speedup 1.0000x reference)
import jax
import jax.numpy as jnp
from jax.experimental import pallas as pl


def kernel(node_embeddings, docstring_embeddings, docstring_mask, edge_sources, edge_dests, edge_types, num_edges, true_indexes, false_indexes, raise_indexes, start_node_indexes, exit_node_indexes, step_limits, W_edge, b_edge, W_ir, b_ir, W_iz, b_iz, W_in, b_in, W_hr, W_hz, W_hn, b_hn, W_out, b_out):
    raise NotImplementedError("write your pallas kernel here")


# speedup vs baseline: 3.4316x; 3.4316x over previous
import functools

import jax
import jax.numpy as jnp
from jax import lax
from jax.experimental import pallas as pl
from jax.experimental.pallas import tpu as pltpu
from jax.experimental.pallas import tpu_sc as plsc

B, N, E, H, NET, L, C = 4, 2048, 16384, 256, 6, 8, 1000

NC, NS = 2, 16
CW = 16
ECORE = E // NC
CHUNK = 512
NCH = ECORE // CHUNK
NPAD = N + 8


def _sc_body(y_hbm, src_hbm, dst_hbm, et_hbm, ne_hbm, out_hbm,
             src_v, dst_v, et_v, gi_v, rows_v, ne_v, acc, sem):
    c = lax.axis_index("c")
    s = lax.axis_index("s")
    iota = lax.iota(jnp.int32, 16)

    def do_batch(b, _):
        def zero_row(r, _):
            acc[pl.ds(r * CW, CW)] = jnp.zeros((CW,), jnp.float32)
            return 0
        lax.fori_loop(0, NPAD, zero_row, 0)
        pltpu.sync_copy(ne_hbm.at[b], ne_v)
        nevec = ne_v[...]

        def do_chunk(k, _):
            eoff = c * ECORE + k * CHUNK
            off = b * E + eoff
            pltpu.sync_copy(src_hbm.at[pl.ds(off, CHUNK)], src_v)
            pltpu.sync_copy(dst_hbm.at[pl.ds(off, CHUNK)], dst_v)
            pltpu.sync_copy(et_hbm.at[pl.ds(off, CHUNK)], et_v)
            for j in range(CHUNK // 16):
                sv = src_v[pl.ds(j * 16, 16)]
                tv = et_v[pl.ds(j * 16, 16)]
                gi_v[pl.ds(j * 16, 16)] = (sv * NET + tv + b * (N * NET)) * (H // CW) + s
            pltpu.async_copy(y_hbm.at[gi_v], rows_v, sem).wait()
            for j in range(CHUNK // 16):
                dv0 = dst_v[pl.ds(j * 16, 16)]
                eid = iota + (eoff + j * 16)
                dvs = jnp.where(eid < nevec, dv0, N) * CW
                for e in range(16):
                    dve = jnp.take_along_axis(dvs, jnp.full((16,), e, jnp.int32), axis=0)
                    plsc.addupdate_scatter(acc, [dve + iota], rows_v[j * 16 + e])
            return 0
        lax.fori_loop(0, NCH, do_chunk, 0)
        base = ((c * B + b) * NS + s) * (N * CW)
        pltpu.sync_copy(acc.at[pl.ds(0, N * CW)], out_hbm.at[pl.ds(base, N * CW)])
        return 0

    lax.fori_loop(0, B, do_batch, 0)


_sc_segsum = pl.kernel(
    _sc_body,
    out_type=jax.ShapeDtypeStruct((NC * B * NS * N * CW,), jnp.float32),
    mesh=plsc.VectorSubcoreMesh(core_axis_name="c", subcore_axis_name="s"),
    compiler_params=pltpu.CompilerParams(needs_layout_passes=False,
                                         use_tc_tiling_on_sc=False),
    scratch_types=[
        pltpu.VMEM((CHUNK,), jnp.int32),
        pltpu.VMEM((CHUNK,), jnp.int32),
        pltpu.VMEM((CHUNK,), jnp.int32),
        pltpu.VMEM((CHUNK,), jnp.int32),
        pltpu.VMEM((CHUNK, CW), jnp.float32),
        pltpu.VMEM((16,), jnp.int32),
        pltpu.VMEM((NPAD * CW,), jnp.float32),
        pltpu.SemaphoreType.DMA,
    ],
)

BN = 512


def _et_body(x_ref, w_ref, be_ref, y_ref):
    y_ref[...] = (jnp.dot(x_ref[0], w_ref[...],
                          preferred_element_type=jnp.float32) + be_ref[...])[None]


def _edge_transform(x, w_edge, be2d):
    return pl.pallas_call(
        _et_body,
        grid=(B, N // BN),
        in_specs=[pl.BlockSpec((1, BN, H), lambda b, i: (b, i, 0)),
                  pl.BlockSpec((H, NET * H), lambda b, i: (0, 0)),
                  pl.BlockSpec((1, NET * H), lambda b, i: (0, 0))],
        out_specs=pl.BlockSpec((1, BN, NET * H), lambda b, i: (b, i, 0)),
        out_shape=jax.ShapeDtypeStruct((B, N, NET * H), jnp.float32),
    )(x, w_edge, be2d)


def _gru_body(x_ref, p_ref, wi_ref, wh_ref, bi_ref, bhn_ref, o_ref):
    xb = x_ref[0]
    p = p_ref[0, 0] + p_ref[1, 0]
    xi = jnp.dot(xb, wi_ref[...], preferred_element_type=jnp.float32) + bi_ref[...]
    ph = jnp.dot(p, wh_ref[...], preferred_element_type=jnp.float32)
    r = jax.nn.sigmoid(xi[:, :H] + ph[:, :H])
    z = jax.nn.sigmoid(xi[:, H:2 * H] + ph[:, H:2 * H])
    n = jnp.tanh(xi[:, 2 * H:] + r * (ph[:, 2 * H:] + bhn_ref[...]))
    o_ref[...] = ((1.0 - z) * n + z * p)[None]


def _gru(x, parts, wi, wh, bi, bhn):
    return pl.pallas_call(
        _gru_body,
        grid=(B, N // BN),
        in_specs=[pl.BlockSpec((1, BN, H), lambda b, i: (b, i, 0)),
                  pl.BlockSpec((NC, 1, BN, H), lambda b, i: (0, b, i, 0)),
                  pl.BlockSpec((H, 3 * H), lambda b, i: (0, 0)),
                  pl.BlockSpec((H, 3 * H), lambda b, i: (0, 0)),
                  pl.BlockSpec((1, 3 * H), lambda b, i: (0, 0)),
                  pl.BlockSpec((1, H), lambda b, i: (0, 0))],
        out_specs=pl.BlockSpec((1, BN, H), lambda b, i: (b, i, 0)),
        out_shape=jax.ShapeDtypeStruct((B, N, H), jnp.float32),
    )(x, parts, wi, wh, bi, bhn)


def _out_body(eidx_ref, h_ref, w_ref, b_ref, o_ref):
    o_ref[...] = (jnp.dot(h_ref[0], w_ref[...],
                          preferred_element_type=jnp.float32) + b_ref[...])[None]


def _readout(exit_idx, x, w_out, b2d):
    return pl.pallas_call(
        _out_body,
        grid_spec=pltpu.PrefetchScalarGridSpec(
            num_scalar_prefetch=1,
            grid=(B,),
            in_specs=[pl.BlockSpec((1, 1, H), lambda b, eidx: (b * N + eidx[b], 0, 0)),
                      pl.BlockSpec((H, C), lambda b, eidx: (0, 0)),
                      pl.BlockSpec((1, C), lambda b, eidx: (0, 0))],
            out_specs=pl.BlockSpec((1, 1, C), lambda b, eidx: (b, 0, 0)),
        ),
        out_shape=jax.ShapeDtypeStruct((B, 1, C), jnp.float32),
    )(exit_idx, x.reshape(B * N, 1, H), w_out, b2d).reshape(B, C)


def kernel(node_embeddings, docstring_embeddings, docstring_mask, edge_sources,
           edge_dests, edge_types, num_edges, true_indexes, false_indexes,
           raise_indexes, start_node_indexes, exit_node_indexes, step_limits,
           W_edge, b_edge, W_ir, b_ir, W_iz, b_iz, W_in, b_in,
           W_hr, W_hz, W_hn, b_hn, W_out, b_out):
    src = edge_sources.reshape(-1).astype(jnp.int32)
    dst = edge_dests.reshape(-1).astype(jnp.int32)
    et = edge_types.reshape(-1).astype(jnp.int32)
    ne_b = jnp.broadcast_to(num_edges.astype(jnp.int32), (B, 16))
    wi = jnp.concatenate([W_ir, W_iz, W_in], axis=1)
    wh = jnp.concatenate([W_hr, W_hz, W_hn], axis=1)
    bi = jnp.concatenate([b_ir, b_iz, b_in])[None]
    bhn = b_hn[None]
    be2d = b_edge[None]
    exit_idx = exit_node_indexes.reshape(B).astype(jnp.int32)

    x = node_embeddings
    for _ in range(L):
        y = _edge_transform(x, W_edge, be2d)
        pflat = _sc_segsum(y.reshape(B * N * NET * (H // CW), CW), src, dst, et, ne_b)
        parts = (pflat.reshape(NC, B, NS, N, CW)
                 .transpose(0, 1, 3, 2, 4).reshape(NC, B, N, H))
        x = _gru(x, parts, wi, wh, bi, bhn)
    return _readout(exit_idx, x, W_out, b_out[None])
